# baseline (speedup 1.0000x reference)
import jax
import jax.numpy as jnp
from jax import lax
from jax.experimental import pallas as pl
from jax.experimental.pallas import tpu as pltpu
from jax.experimental.pallas import tpu_sc as plsc

A = 9
H = 50
W = 50
HW = H * W
N = A * HW
NPAD = 22528
NWORK = 16
SL = NPAD // NWORK
NCH = SL // 16
PRE = 20
POST = 10
NMS_T = 0.7
NEG = -1e9
NEGINF = -3.0e38
BIG = 3.0e38

_f32 = jnp.float32
_i32 = jnp.int32


def _iota16():
    return lax.broadcasted_iota(_i32, (16,), 0)


def _trunc(x):
    return x.astype(_i32).astype(_f32)


def _splat_i(x):
    return jnp.full((16,), x, _i32)


def _splat_f(x):
    return jnp.full((16,), x, _f32)


def _perm(v, idx):
    return v.at[idx].get(mode="promise_in_bounds")


def _bcast_max(v):
    it = _iota16()
    for s in (1, 2, 4, 8):
        v = jnp.maximum(v, _perm(v, it ^ s))
    return v


def _bcast_min(v):
    it = _iota16()
    for s in (1, 2, 4, 8):
        v = jnp.minimum(v, _perm(v, it ^ s))
    return v


def _sc_body(fg_hbm, dx_hbm, dy_hbm, dw_hbm, dh_hbm, im_hbm, out_hbm,
             sfg, sdx, sdy, sdw, sdh, cfg, cF, cx1, cy1, cx2, cy2, imv,
             lsc, lF, lx1, ly1, lx2, ly2,
             shsc, shF, shx1, shy1, shx2, shy2,
             gsc, gF, gx1, gy1, gx2, gy2,
             tsc2, tx1, ty1, tx2, ty2, tar, tkeep, fsv, stage):
    wid = lax.axis_index("s")
    base = wid * SL
    it = _iota16()
    lane0 = it == 0

    pltpu.sync_copy(fg_hbm.at[pl.ds(base, SL)], sfg)
    pltpu.sync_copy(dx_hbm.at[pl.ds(base, SL)], sdx)
    pltpu.sync_copy(dy_hbm.at[pl.ds(base, SL)], sdy)
    pltpu.sync_copy(dw_hbm.at[pl.ds(base, SL)], sdw)
    pltpu.sync_copy(dh_hbm.at[pl.ds(base, SL)], sdh)
    pltpu.sync_copy(im_hbm, imv)

    imh = imv[pl.ds(0, 16)]
    imw = imv[pl.ds(16, 16)]
    imsc = imv[pl.ds(32, 16)]
    minsz = 5.0 * imsc

    basef = base.astype(_f32)

    def decode(t, _):
        off = t * 16
        g = (base + off + it).astype(_f32)
        af = _trunc(g / 2500.0)
        pf = g - af * 2500.0
        Ff = pf * 9.0 + af
        hh = _trunc(pf / 50.0)
        ww = pf - hh * 50.0
        ridx = _trunc(af / 3.0)
        sidx = af - ridx * 3.0
        wr = jnp.where(ridx == 0.0, 23.0, jnp.where(ridx == 1.0, 16.0, 11.0))
        hr = jnp.where(ridx == 0.0, 12.0, jnp.where(ridx == 1.0, 16.0, 22.0))
        sc = jnp.where(sidx == 0.0, 8.0, jnp.where(sidx == 1.0, 16.0, 32.0))
        wa = wr * sc
        ha = hr * sc
        ctx = 8.0 + ww * 16.0
        cty = 8.0 + hh * 16.0
        dxv = sdx[pl.ds(off, 16)]
        dyv = sdy[pl.ds(off, 16)]
        dwv = jnp.clip(sdw[pl.ds(off, 16)], -10.0, 4.135)
        dhv = jnp.clip(sdh[pl.ds(off, 16)], -10.0, 4.135)
        pcx = dxv * wa + ctx
        pcy = dyv * ha + cty
        pwv = jnp.exp(dwv) * wa
        phv = jnp.exp(dhv) * ha
        x1 = jnp.clip(pcx - 0.5 * pwv, 0.0, imw - 1.0)
        x2 = jnp.clip(pcx + 0.5 * pwv, 0.0, imw - 1.0)
        y1 = jnp.clip(pcy - 0.5 * phv, 0.0, imh - 1.0)
        y2 = jnp.clip(pcy + 0.5 * phv, 0.0, imh - 1.0)
        wsv = x2 - x1 + 1.0
        hsv = y2 - y1 + 1.0
        valid = (wsv >= minsz) & (hsv >= minsz)
        fgm = jnp.where(valid, sfg[pl.ds(off, 16)], NEG)
        fgm = jnp.where(g < float(N), fgm, NEGINF)
        cfg[pl.ds(off, 16)] = fgm
        cF[pl.ds(off, 16)] = Ff
        cx1[pl.ds(off, 16)] = x1
        cy1[pl.ds(off, 16)] = y1
        cx2[pl.ds(off, 16)] = x2
        cy2[pl.ds(off, 16)] = y2
        return 0

    lax.fori_loop(0, NCH, decode, 0)

    lsc[pl.ds(0, 16)] = _splat_f(NEGINF)
    lsc[pl.ds(16, 16)] = _splat_f(NEGINF)
    lF[pl.ds(0, 16)] = _splat_f(0.0)
    lF[pl.ds(16, 16)] = _splat_f(0.0)
    for ref in (lx1, ly1, lx2, ly2):
        ref[pl.ds(0, 16)] = _splat_f(0.0)
        ref[pl.ds(16, 16)] = _splat_f(0.0)

    def pick(r, _):
        M = _splat_f(NEGINF)
        FM = _splat_f(0.0)
        for k in range(NCH):
            v = cfg[pl.ds(k * 16, 16)]
            f = cF[pl.ds(k * 16, 16)]
            better = (v > M) | ((v == M) & (f < FM))
            M = jnp.where(better, v, M)
            FM = jnp.where(better, f, FM)
        m = _bcast_max(M)
        fm = _bcast_min(jnp.where(M == m, FM, BIG))
        pf = _trunc(fm / 9.0)
        af = fm - pf * 9.0
        g = af * 2500.0 + pf
        oidx = (g - basef).astype(_i32)
        plsc.store_scatter(cfg, [oidx], _splat_f(NEGINF), mask=lane0)
        ridx = _splat_i(r)
        plsc.store_scatter(lsc, [ridx], m, mask=lane0)
        plsc.store_scatter(lF, [ridx], fm, mask=lane0)
        plsc.store_scatter(lx1, [ridx], plsc.load_gather(cx1, [oidx]), mask=lane0)
        plsc.store_scatter(ly1, [ridx], plsc.load_gather(cy1, [oidx]), mask=lane0)
        plsc.store_scatter(lx2, [ridx], plsc.load_gather(cx2, [oidx]), mask=lane0)
        plsc.store_scatter(ly2, [ridx], plsc.load_gather(cy2, [oidx]), mask=lane0)
        return 0

    lax.fori_loop(0, PRE, pick, 0)

    pltpu.sync_copy(lsc, shsc.at[pl.ds(wid * 32, 32)])
    pltpu.sync_copy(lF, shF.at[pl.ds(wid * 32, 32)])
    pltpu.sync_copy(lx1, shx1.at[pl.ds(wid * 32, 32)])
    pltpu.sync_copy(ly1, shy1.at[pl.ds(wid * 32, 32)])
    pltpu.sync_copy(lx2, shx2.at[pl.ds(wid * 32, 32)])
    pltpu.sync_copy(ly2, shy2.at[pl.ds(wid * 32, 32)])
    plsc.subcore_barrier()

    @pl.when(wid == 0)
    def _merge():
        pltpu.sync_copy(shsc, gsc)
        pltpu.sync_copy(shF, gF)
        pltpu.sync_copy(shx1, gx1)
        pltpu.sync_copy(shy1, gy1)
        pltpu.sync_copy(shx2, gx2)
        pltpu.sync_copy(shy2, gy2)

        for c in range(2):
            tsc2[pl.ds(c * 16, 16)] = _splat_f(NEGINF)
            for ref in (tx1, ty1, tx2, ty2, tar):
                ref[pl.ds(c * 16, 16)] = _splat_f(0.0)

        NG = 2 * NWORK

        def gpick(r, _):
            M = _splat_f(NEGINF)
            FM = _splat_f(0.0)
            S = _splat_i(0)
            for k in range(NG):
                v = gsc[pl.ds(k * 16, 16)]
                f = gF[pl.ds(k * 16, 16)]
                s = _splat_i(k * 16) + it
                better = (v > M) | ((v == M) & (f < FM))
                M = jnp.where(better, v, M)
                FM = jnp.where(better, f, FM)
                S = jnp.where(better, s, S)
            m = _bcast_max(M)
            fm = _bcast_min(jnp.where(M == m, FM, BIG))
            sidx = _bcast_min(jnp.where((M == m) & (FM == fm), S, 2147483647))
            plsc.store_scatter(gsc, [sidx], _splat_f(NEGINF), mask=lane0)
            ridx = _splat_i(r)
            bx1 = plsc.load_gather(gx1, [sidx])
            by1 = plsc.load_gather(gy1, [sidx])
            bx2 = plsc.load_gather(gx2, [sidx])
            by2 = plsc.load_gather(gy2, [sidx])
            plsc.store_scatter(tsc2, [ridx], m, mask=lane0)
            plsc.store_scatter(tx1, [ridx], bx1, mask=lane0)
            plsc.store_scatter(ty1, [ridx], by1, mask=lane0)
            plsc.store_scatter(tx2, [ridx], bx2, mask=lane0)
            plsc.store_scatter(ty2, [ridx], by2, mask=lane0)
            plsc.store_scatter(
                tar, [ridx],
                (bx2 - bx1 + 1.0) * (by2 - by1 + 1.0), mask=lane0)
            return 0

        lax.fori_loop(0, PRE, gpick, 0)

        tkeep[pl.ds(0, 16)] = _splat_f(1.0)
        tkeep[pl.ds(16, 16)] = jnp.where(it + 16 < PRE, 1.0, 0.0)

        def nms(i, _):
            iidx = _splat_i(i)
            ki = plsc.load_gather(tkeep, [iidx])
            ai = plsc.load_gather(tar, [iidx])
            x1i = plsc.load_gather(tx1, [iidx])
            y1i = plsc.load_gather(ty1, [iidx])
            x2i = plsc.load_gather(tx2, [iidx])
            y2i = plsc.load_gather(ty2, [iidx])
            for c in range(2):
                sv = _splat_i(c * 16) + it
                a1 = tx1[pl.ds(c * 16, 16)]
                b1 = ty1[pl.ds(c * 16, 16)]
                a2 = tx2[pl.ds(c * 16, 16)]
                b2 = ty2[pl.ds(c * 16, 16)]
                ar = tar[pl.ds(c * 16, 16)]
                xx1 = jnp.maximum(x1i, a1)
                yy1 = jnp.maximum(y1i, b1)
                xx2 = jnp.minimum(x2i, a2)
                yy2 = jnp.minimum(y2i, b2)
                iw = jnp.maximum(xx2 - xx1 + 1.0, 0.0)
                ih = jnp.maximum(yy2 - yy1 + 1.0, 0.0)
                inter = iw * ih
                iou = inter / (ai + ar - inter)
                kv = tkeep[pl.ds(c * 16, 16)]
                sup = (iou > NMS_T) & (sv > iidx) & (ki > 0.0)
                tkeep[pl.ds(c * 16, 16)] = jnp.where(sup, 0.0, kv)
            return 0

        lax.fori_loop(0, PRE, nms, 0)

        for c in range(2):
            sv = _splat_i(c * 16) + it
            kv = tkeep[pl.ds(c * 16, 16)]
            fs = jnp.where(kv > 0.0, tsc2[pl.ds(c * 16, 16)], NEG)
            fsv[pl.ds(c * 16, 16)] = jnp.where(sv < PRE, fs, NEGINF)

        for c in range(4):
            stage[pl.ds(c * 16, 16)] = _splat_f(0.0)

        def outp(j, _):
            M = _splat_f(NEGINF)
            S = _splat_i(0)
            for k in range(2):
                v = fsv[pl.ds(k * 16, 16)]
                s = _splat_i(k * 16) + it
                better = (v > M) | ((v == M) & (s < S))
                M = jnp.where(better, v, M)
                S = jnp.where(better, s, S)
            m = _bcast_max(M)
            sidx = _bcast_min(jnp.where(M == m, S, 2147483647))
            plsc.store_scatter(fsv, [sidx], _splat_f(NEGINF), mask=lane0)
            bx1 = plsc.load_gather(tx1, [sidx])
            by1 = plsc.load_gather(ty1, [sidx])
            bx2 = plsc.load_gather(tx2, [sidx])
            by2 = plsc.load_gather(ty2, [sidx])
            val = jnp.where(it == 0, bx1,
                            jnp.where(it == 1, by1,
                                      jnp.where(it == 2, bx2, by2)))
            oidx = _splat_i(4 * j) + it
            plsc.store_scatter(stage, [oidx], val, mask=it < 4)
            return 0

        lax.fori_loop(0, POST, outp, 0)
        pltpu.sync_copy(stage, out_hbm)


def kernel(scores, bbox_deltas, im_info):
    fg = scores[0, A:].reshape(-1)
    bd = bbox_deltas[0].reshape(A, 4, HW)
    zpad = jnp.zeros((NPAD - N,), _f32)
    fgp = jnp.concatenate([fg, zpad])
    dxp = jnp.concatenate([bd[:, 0, :].reshape(-1), zpad])
    dyp = jnp.concatenate([bd[:, 1, :].reshape(-1), zpad])
    dwp = jnp.concatenate([bd[:, 2, :].reshape(-1), zpad])
    dhp = jnp.concatenate([bd[:, 3, :].reshape(-1), zpad])
    imf = jnp.repeat(im_info.reshape(-1), 16)

    mesh = plsc.VectorSubcoreMesh(
        core_axis_name="c", subcore_axis_name="s", num_cores=1)
    vm = pltpu.VMEM
    out = pl.kernel(
        _sc_body,
        out_type=jax.ShapeDtypeStruct((64,), _f32),
        mesh=mesh,
        scratch_types=[
            vm((SL,), _f32), vm((SL,), _f32), vm((SL,), _f32),
            vm((SL,), _f32), vm((SL,), _f32),
            vm((SL,), _f32), vm((SL,), _f32), vm((SL,), _f32),
            vm((SL,), _f32), vm((SL,), _f32), vm((SL,), _f32),
            vm((48,), _f32),
            vm((32,), _f32), vm((32,), _f32), vm((32,), _f32),
            vm((32,), _f32), vm((32,), _f32), vm((32,), _f32),
            pltpu.VMEM_SHARED((512,), _f32), pltpu.VMEM_SHARED((512,), _f32),
            pltpu.VMEM_SHARED((512,), _f32), pltpu.VMEM_SHARED((512,), _f32),
            pltpu.VMEM_SHARED((512,), _f32), pltpu.VMEM_SHARED((512,), _f32),
            vm((512,), _f32), vm((512,), _f32), vm((512,), _f32),
            vm((512,), _f32), vm((512,), _f32), vm((512,), _f32),
            vm((32,), _f32), vm((32,), _f32), vm((32,), _f32),
            vm((32,), _f32), vm((32,), _f32), vm((32,), _f32),
            vm((32,), _f32), vm((32,), _f32),
            vm((64,), _f32),
        ],
        compiler_params=pltpu.CompilerParams(needs_layout_passes=False),
    )(fgp, dxp, dyp, dwp, dhp, imf)
    return out[:40].reshape(POST, 4)


# speedup vs baseline: 1.0937x; 1.0937x over previous
import jax
import jax.numpy as jnp
from jax import lax
from jax.experimental import pallas as pl
from jax.experimental.pallas import tpu as pltpu
from jax.experimental.pallas import tpu_sc as plsc

A = 9
H = 50
W = 50
HW = H * W
N = A * HW
NPAD = 22528
NWORK = 16
SL = NPAD // NWORK
NCH = SL // 16
PRE = 20
POST = 10
NMS_T = 0.7
NEG = -1e9
NEGINF = -3.0e38
BIG = 3.0e38

_f32 = jnp.float32
_i32 = jnp.int32


def _iota16():
    return lax.broadcasted_iota(_i32, (16,), 0)


def _trunc(x):
    return x.astype(_i32).astype(_f32)


def _splat_i(x):
    return jnp.full((16,), x, _i32)


def _splat_f(x):
    return jnp.full((16,), x, _f32)


def _perm(v, idx):
    return v.at[idx].get(mode="promise_in_bounds")


def _bcast_max(v):
    it = _iota16()
    for s in (1, 2, 4, 8):
        v = jnp.maximum(v, _perm(v, it ^ s))
    return v


def _bcast_min(v):
    it = _iota16()
    for s in (1, 2, 4, 8):
        v = jnp.minimum(v, _perm(v, it ^ s))
    return v


def _sc_body(x_hbm, im_hbm, out_hbm,
             sall, cfg, cF, cx1, cy1, cx2, cy2, imv, lpk, shpk, gpk,
             tsc2, tx1, ty1, tx2, ty2, tar, tkeep, fsv, stage, dsem):
    wid = lax.axis_index("s")
    it = _iota16()
    lane0 = it == 0

    cp = pltpu.async_copy(x_hbm.at[pl.ds(wid * (5 * SL), 5 * SL)], sall, dsem)
    pltpu.sync_copy(im_hbm, imv)
    cp.wait()

    imh = imv[pl.ds(0, 16)]
    imw = imv[pl.ds(16, 16)]
    imsc = imv[pl.ds(32, 16)]
    minsz = 5.0 * imsc

    base = wid * SL
    basef = base.astype(_f32)

    def decode(t, _):
        for u in range(4):
            off = t * 64 + u * 16
            g = (base + off + it).astype(_f32)
            af = _trunc(g / 2500.0)
            pf = g - af * 2500.0
            Ff = pf * 9.0 + af
            hh = _trunc(pf / 50.0)
            ww = pf - hh * 50.0
            ridx = _trunc(af / 3.0)
            sidx = af - ridx * 3.0
            wr = jnp.where(ridx == 0.0, 23.0,
                           jnp.where(ridx == 1.0, 16.0, 11.0))
            hr = jnp.where(ridx == 0.0, 12.0,
                           jnp.where(ridx == 1.0, 16.0, 22.0))
            sc = jnp.where(sidx == 0.0, 8.0,
                           jnp.where(sidx == 1.0, 16.0, 32.0))
            wa = wr * sc
            ha = hr * sc
            ctx = 8.0 + ww * 16.0
            cty = 8.0 + hh * 16.0
            dxv = sall[pl.ds(SL + off, 16)]
            dyv = sall[pl.ds(2 * SL + off, 16)]
            dwv = jnp.clip(sall[pl.ds(3 * SL + off, 16)], -10.0, 4.135)
            dhv = jnp.clip(sall[pl.ds(4 * SL + off, 16)], -10.0, 4.135)
            pcx = dxv * wa + ctx
            pcy = dyv * ha + cty
            pwv = jnp.exp(dwv) * wa
            phv = jnp.exp(dhv) * ha
            x1 = jnp.clip(pcx - 0.5 * pwv, 0.0, imw - 1.0)
            x2 = jnp.clip(pcx + 0.5 * pwv, 0.0, imw - 1.0)
            y1 = jnp.clip(pcy - 0.5 * phv, 0.0, imh - 1.0)
            y2 = jnp.clip(pcy + 0.5 * phv, 0.0, imh - 1.0)
            wsv = x2 - x1 + 1.0
            hsv = y2 - y1 + 1.0
            valid = (wsv >= minsz) & (hsv >= minsz)
            fgm = jnp.where(valid, sall[pl.ds(off, 16)], NEG)
            fgm = jnp.where(g < float(N), fgm, NEGINF)
            cfg[pl.ds(off, 16)] = fgm
            cF[pl.ds(off, 16)] = Ff
            cx1[pl.ds(off, 16)] = x1
            cy1[pl.ds(off, 16)] = y1
            cx2[pl.ds(off, 16)] = x2
            cy2[pl.ds(off, 16)] = y2
        return 0

    lax.fori_loop(0, NCH // 4, decode, 0)

    lpk[pl.ds(0, 16)] = _splat_f(NEGINF)
    lpk[pl.ds(16, 16)] = _splat_f(NEGINF)
    for c in range(2, 12):
        lpk[pl.ds(c * 16, 16)] = _splat_f(0.0)

    def pick(r, _):
        M = _splat_f(NEGINF)
        FM = _splat_f(0.0)
        for k in range(NCH):
            v = cfg[pl.ds(k * 16, 16)]
            f = cF[pl.ds(k * 16, 16)]
            better = (v > M) | ((v == M) & (f < FM))
            M = jnp.where(better, v, M)
            FM = jnp.where(better, f, FM)
        m = _bcast_max(M)
        fm = _bcast_min(jnp.where(M == m, FM, BIG))
        pf = _trunc(fm / 9.0)
        af = fm - pf * 9.0
        g = af * 2500.0 + pf
        oidx = (g - basef).astype(_i32)
        plsc.store_scatter(cfg, [oidx], _splat_f(NEGINF), mask=lane0)
        ridx = _splat_i(r)
        plsc.store_scatter(lpk, [ridx], m, mask=lane0)
        plsc.store_scatter(lpk, [ridx + 32], fm, mask=lane0)
        plsc.store_scatter(lpk, [ridx + 64],
                           plsc.load_gather(cx1, [oidx]), mask=lane0)
        plsc.store_scatter(lpk, [ridx + 96],
                           plsc.load_gather(cy1, [oidx]), mask=lane0)
        plsc.store_scatter(lpk, [ridx + 128],
                           plsc.load_gather(cx2, [oidx]), mask=lane0)
        plsc.store_scatter(lpk, [ridx + 160],
                           plsc.load_gather(cy2, [oidx]), mask=lane0)
        return 0

    lax.fori_loop(0, PRE, pick, 0)

    pltpu.sync_copy(lpk, shpk.at[pl.ds(wid * 192, 192)])
    plsc.subcore_barrier()

    @pl.when(wid == 0)
    def _merge():
        pltpu.sync_copy(shpk, gpk)

        for c in range(2):
            tsc2[pl.ds(c * 16, 16)] = _splat_f(NEGINF)
            for ref in (tx1, ty1, tx2, ty2, tar):
                ref[pl.ds(c * 16, 16)] = _splat_f(0.0)

        def gpick(r, _):
            M = _splat_f(NEGINF)
            FM = _splat_f(0.0)
            S = _splat_i(0)
            for w in range(NWORK):
                for hh2 in range(2):
                    b = w * 192 + hh2 * 16
                    v = gpk[pl.ds(b, 16)]
                    f = gpk[pl.ds(b + 32, 16)]
                    s = _splat_i(b) + it
                    better = (v > M) | ((v == M) & (f < FM))
                    M = jnp.where(better, v, M)
                    FM = jnp.where(better, f, FM)
                    S = jnp.where(better, s, S)
            m = _bcast_max(M)
            fm = _bcast_min(jnp.where(M == m, FM, BIG))
            sidx = _bcast_min(jnp.where((M == m) & (FM == fm), S, 2147483647))
            plsc.store_scatter(gpk, [sidx], _splat_f(NEGINF), mask=lane0)
            ridx = _splat_i(r)
            bx1 = plsc.load_gather(gpk, [sidx + 64])
            by1 = plsc.load_gather(gpk, [sidx + 96])
            bx2 = plsc.load_gather(gpk, [sidx + 128])
            by2 = plsc.load_gather(gpk, [sidx + 160])
            plsc.store_scatter(tsc2, [ridx], m, mask=lane0)
            plsc.store_scatter(tx1, [ridx], bx1, mask=lane0)
            plsc.store_scatter(ty1, [ridx], by1, mask=lane0)
            plsc.store_scatter(tx2, [ridx], bx2, mask=lane0)
            plsc.store_scatter(ty2, [ridx], by2, mask=lane0)
            plsc.store_scatter(
                tar, [ridx],
                (bx2 - bx1 + 1.0) * (by2 - by1 + 1.0), mask=lane0)
            return 0

        lax.fori_loop(0, PRE, gpick, 0)

        tkeep[pl.ds(0, 16)] = _splat_f(1.0)
        tkeep[pl.ds(16, 16)] = jnp.where(it + 16 < PRE, 1.0, 0.0)

        def nms(i, _):
            iidx = _splat_i(i)
            ki = plsc.load_gather(tkeep, [iidx])
            ai = plsc.load_gather(tar, [iidx])
            x1i = plsc.load_gather(tx1, [iidx])
            y1i = plsc.load_gather(ty1, [iidx])
            x2i = plsc.load_gather(tx2, [iidx])
            y2i = plsc.load_gather(ty2, [iidx])
            for c in range(2):
                sv = _splat_i(c * 16) + it
                a1 = tx1[pl.ds(c * 16, 16)]
                b1 = ty1[pl.ds(c * 16, 16)]
                a2 = tx2[pl.ds(c * 16, 16)]
                b2 = ty2[pl.ds(c * 16, 16)]
                ar = tar[pl.ds(c * 16, 16)]
                xx1 = jnp.maximum(x1i, a1)
                yy1 = jnp.maximum(y1i, b1)
                xx2 = jnp.minimum(x2i, a2)
                yy2 = jnp.minimum(y2i, b2)
                iw = jnp.maximum(xx2 - xx1 + 1.0, 0.0)
                ih = jnp.maximum(yy2 - yy1 + 1.0, 0.0)
                inter = iw * ih
                iou = inter / (ai + ar - inter)
                kv = tkeep[pl.ds(c * 16, 16)]
                sup = (iou > NMS_T) & (sv > iidx) & (ki > 0.0)
                tkeep[pl.ds(c * 16, 16)] = jnp.where(sup, 0.0, kv)
            return 0

        lax.fori_loop(0, PRE, nms, 0)

        for c in range(2):
            sv = _splat_i(c * 16) + it
            kv = tkeep[pl.ds(c * 16, 16)]
            fs = jnp.where(kv > 0.0, tsc2[pl.ds(c * 16, 16)], NEG)
            fsv[pl.ds(c * 16, 16)] = jnp.where(sv < PRE, fs, NEGINF)

        for c in range(4):
            stage[pl.ds(c * 16, 16)] = _splat_f(0.0)

        def outp(j, _):
            M = _splat_f(NEGINF)
            S = _splat_i(0)
            for k in range(2):
                v = fsv[pl.ds(k * 16, 16)]
                s = _splat_i(k * 16) + it
                better = (v > M) | ((v == M) & (s < S))
                M = jnp.where(better, v, M)
                S = jnp.where(better, s, S)
            m = _bcast_max(M)
            sidx = _bcast_min(jnp.where(M == m, S, 2147483647))
            plsc.store_scatter(fsv, [sidx], _splat_f(NEGINF), mask=lane0)
            bx1 = plsc.load_gather(tx1, [sidx])
            by1 = plsc.load_gather(ty1, [sidx])
            bx2 = plsc.load_gather(tx2, [sidx])
            by2 = plsc.load_gather(ty2, [sidx])
            val = jnp.where(it == 0, bx1,
                            jnp.where(it == 1, by1,
                                      jnp.where(it == 2, bx2, by2)))
            oidx = _splat_i(4 * j) + it
            plsc.store_scatter(stage, [oidx], val, mask=it < 4)
            return 0

        lax.fori_loop(0, POST, outp, 0)
        pltpu.sync_copy(stage, out_hbm)


def kernel(scores, bbox_deltas, im_info):
    fg = scores[0, A:].reshape(-1)
    bd = bbox_deltas[0].reshape(A, 4, HW)
    zpad = jnp.zeros((NPAD - N,), _f32)
    comps = [jnp.concatenate([fg, zpad])]
    for j in range(4):
        comps.append(jnp.concatenate([bd[:, j, :].reshape(-1), zpad]))
    xall = jnp.stack(comps).reshape(5, NWORK, SL).transpose(1, 0, 2).reshape(-1)
    imf = jnp.repeat(im_info.reshape(-1), 16)

    mesh = plsc.VectorSubcoreMesh(
        core_axis_name="c", subcore_axis_name="s", num_cores=1)
    vm = pltpu.VMEM
    out = pl.kernel(
        _sc_body,
        out_type=jax.ShapeDtypeStruct((64,), _f32),
        mesh=mesh,
        scratch_types=[
            vm((5 * SL,), _f32),
            vm((SL,), _f32), vm((SL,), _f32),
            vm((SL,), _f32), vm((SL,), _f32),
            vm((SL,), _f32), vm((SL,), _f32),
            vm((48,), _f32),
            vm((192,), _f32),
            pltpu.VMEM_SHARED((3072,), _f32),
            vm((3072,), _f32),
            vm((32,), _f32), vm((32,), _f32), vm((32,), _f32),
            vm((32,), _f32), vm((32,), _f32), vm((32,), _f32),
            vm((32,), _f32), vm((32,), _f32),
            vm((64,), _f32),
            pltpu.SemaphoreType.DMA,
        ],
        compiler_params=pltpu.CompilerParams(needs_layout_passes=False),
    )(xall, imf)
    return out[:40].reshape(POST, 4)


# speedup vs baseline: 1.1723x; 1.0719x over previous
import jax
import jax.numpy as jnp
from jax import lax
from jax.experimental import pallas as pl
from jax.experimental.pallas import tpu as pltpu
from jax.experimental.pallas import tpu_sc as plsc

A = 9
H = 50
W = 50
HW = H * W
N = A * HW
NPAD = 22528
NWORK = 16
SL = NPAD // NWORK
NCH = SL // 16
PRE = 20
POST = 10
NMS_T = 0.7
NEG = -1e9
NEGINF = -3.0e38
BIG = 3.0e38

_f32 = jnp.float32
_i32 = jnp.int32


def _iota16():
    return lax.broadcasted_iota(_i32, (16,), 0)


def _trunc(x):
    return x.astype(_i32).astype(_f32)


def _splat_i(x):
    return jnp.full((16,), x, _i32)


def _splat_f(x):
    return jnp.full((16,), x, _f32)


def _perm(v, idx):
    return v.at[idx].get(mode="promise_in_bounds")


def _bcast_max(v):
    it = _iota16()
    for s in (1, 2, 4, 8):
        v = jnp.maximum(v, _perm(v, it ^ s))
    return v


def _bcast_min(v):
    it = _iota16()
    for s in (1, 2, 4, 8):
        v = jnp.minimum(v, _perm(v, it ^ s))
    return v


def _sc_body(x_hbm, im_hbm, out_hbm,
             sall, cfg, cF, cx1, cy1, cx2, cy2, imv, lpk, shpk, gpk,
             tsc2, tx1, ty1, tx2, ty2, tar, tkeep, fsv, stage, dsem):
    wid = lax.axis_index("s")
    it = _iota16()
    lane0 = it == 0

    cp = pltpu.async_copy(x_hbm.at[pl.ds(wid * (5 * SL), 5 * SL)], sall, dsem)
    pltpu.sync_copy(im_hbm, imv)
    cp.wait()

    imh = imv[pl.ds(0, 16)]
    imw = imv[pl.ds(16, 16)]
    imsc = imv[pl.ds(32, 16)]
    minsz = 5.0 * imsc

    base = wid * SL
    basef = base.astype(_f32)

    def decode(t, _):
        for u in range(4):
            off = t * 64 + u * 16
            g = (base + off + it).astype(_f32)
            af = _trunc(g / 2500.0)
            pf = g - af * 2500.0
            Ff = pf * 9.0 + af
            hh = _trunc(pf / 50.0)
            ww = pf - hh * 50.0
            ridx = _trunc(af / 3.0)
            sidx = af - ridx * 3.0
            wr = jnp.where(ridx == 0.0, 23.0,
                           jnp.where(ridx == 1.0, 16.0, 11.0))
            hr = jnp.where(ridx == 0.0, 12.0,
                           jnp.where(ridx == 1.0, 16.0, 22.0))
            sc = jnp.where(sidx == 0.0, 8.0,
                           jnp.where(sidx == 1.0, 16.0, 32.0))
            wa = wr * sc
            ha = hr * sc
            ctx = 8.0 + ww * 16.0
            cty = 8.0 + hh * 16.0
            dxv = sall[pl.ds(SL + off, 16)]
            dyv = sall[pl.ds(2 * SL + off, 16)]
            dwv = jnp.clip(sall[pl.ds(3 * SL + off, 16)], -10.0, 4.135)
            dhv = jnp.clip(sall[pl.ds(4 * SL + off, 16)], -10.0, 4.135)
            pcx = dxv * wa + ctx
            pcy = dyv * ha + cty
            pwv = jnp.exp(dwv) * wa
            phv = jnp.exp(dhv) * ha
            x1 = jnp.clip(pcx - 0.5 * pwv, 0.0, imw - 1.0)
            x2 = jnp.clip(pcx + 0.5 * pwv, 0.0, imw - 1.0)
            y1 = jnp.clip(pcy - 0.5 * phv, 0.0, imh - 1.0)
            y2 = jnp.clip(pcy + 0.5 * phv, 0.0, imh - 1.0)
            wsv = x2 - x1 + 1.0
            hsv = y2 - y1 + 1.0
            valid = (wsv >= minsz) & (hsv >= minsz)
            fgm = jnp.where(valid, sall[pl.ds(off, 16)], NEG)
            fgm = jnp.where(g < float(N), fgm, NEGINF)
            cfg[pl.ds(off, 16)] = fgm
            cF[pl.ds(off, 16)] = Ff
            cx1[pl.ds(off, 16)] = x1
            cy1[pl.ds(off, 16)] = y1
            cx2[pl.ds(off, 16)] = x2
            cy2[pl.ds(off, 16)] = y2
        return 0

    lax.fori_loop(0, NCH // 4, decode, 0)

    lpk[pl.ds(0, 16)] = _splat_f(NEGINF)
    lpk[pl.ds(16, 16)] = _splat_f(NEGINF)
    for c in range(2, 12):
        lpk[pl.ds(c * 16, 16)] = _splat_f(0.0)

    def pick(r, _):
        Ms = [_splat_f(NEGINF)] * 4
        FMs = [_splat_f(0.0)] * 4
        for k in range(NCH):
            c4 = k % 4
            v = cfg[pl.ds(k * 16, 16)]
            f = cF[pl.ds(k * 16, 16)]
            better = (v > Ms[c4]) | ((v == Ms[c4]) & (f < FMs[c4]))
            Ms[c4] = jnp.where(better, v, Ms[c4])
            FMs[c4] = jnp.where(better, f, FMs[c4])
        M, FM = Ms[0], FMs[0]
        for c4 in range(1, 4):
            better = (Ms[c4] > M) | ((Ms[c4] == M) & (FMs[c4] < FM))
            M = jnp.where(better, Ms[c4], M)
            FM = jnp.where(better, FMs[c4], FM)
        m = _bcast_max(M)
        fm = _bcast_min(jnp.where(M == m, FM, BIG))
        pf = _trunc(fm / 9.0)
        af = fm - pf * 9.0
        g = af * 2500.0 + pf
        oidx = (g - basef).astype(_i32)
        plsc.store_scatter(cfg, [oidx], _splat_f(NEGINF), mask=lane0)
        ridx = _splat_i(r)
        plsc.store_scatter(lpk, [ridx], m, mask=lane0)
        plsc.store_scatter(lpk, [ridx + 32], fm, mask=lane0)
        plsc.store_scatter(lpk, [ridx + 64],
                           plsc.load_gather(cx1, [oidx]), mask=lane0)
        plsc.store_scatter(lpk, [ridx + 96],
                           plsc.load_gather(cy1, [oidx]), mask=lane0)
        plsc.store_scatter(lpk, [ridx + 128],
                           plsc.load_gather(cx2, [oidx]), mask=lane0)
        plsc.store_scatter(lpk, [ridx + 160],
                           plsc.load_gather(cy2, [oidx]), mask=lane0)
        return 0

    lax.fori_loop(0, PRE, pick, 0)

    pltpu.sync_copy(lpk, shpk.at[pl.ds(wid * 192, 192)])
    plsc.subcore_barrier()

    @pl.when(wid == 0)
    def _merge():
        pltpu.sync_copy(shpk, gpk)

        for c in range(2):
            tsc2[pl.ds(c * 16, 16)] = _splat_f(NEGINF)
            for ref in (tx1, ty1, tx2, ty2, tar):
                ref[pl.ds(c * 16, 16)] = _splat_f(0.0)

        def gpick(r, _):
            Ms = [_splat_f(NEGINF)] * 4
            FMs = [_splat_f(0.0)] * 4
            Ss = [_splat_i(0)] * 4
            ci = 0
            for w in range(NWORK):
                for hh2 in range(2):
                    b = w * 192 + hh2 * 16
                    c4 = ci % 4
                    ci += 1
                    v = gpk[pl.ds(b, 16)]
                    f = gpk[pl.ds(b + 32, 16)]
                    s = _splat_i(b) + it
                    better = (v > Ms[c4]) | ((v == Ms[c4]) & (f < FMs[c4]))
                    Ms[c4] = jnp.where(better, v, Ms[c4])
                    FMs[c4] = jnp.where(better, f, FMs[c4])
                    Ss[c4] = jnp.where(better, s, Ss[c4])
            M, FM, S = Ms[0], FMs[0], Ss[0]
            for c4 in range(1, 4):
                better = (Ms[c4] > M) | ((Ms[c4] == M) & (FMs[c4] < FM))
                M = jnp.where(better, Ms[c4], M)
                FM = jnp.where(better, FMs[c4], FM)
                S = jnp.where(better, Ss[c4], S)
            m = _bcast_max(M)
            fm = _bcast_min(jnp.where(M == m, FM, BIG))
            sidx = _bcast_min(jnp.where((M == m) & (FM == fm), S, 2147483647))
            plsc.store_scatter(gpk, [sidx], _splat_f(NEGINF), mask=lane0)
            ridx = _splat_i(r)
            bx1 = plsc.load_gather(gpk, [sidx + 64])
            by1 = plsc.load_gather(gpk, [sidx + 96])
            bx2 = plsc.load_gather(gpk, [sidx + 128])
            by2 = plsc.load_gather(gpk, [sidx + 160])
            plsc.store_scatter(tsc2, [ridx], m, mask=lane0)
            plsc.store_scatter(tx1, [ridx], bx1, mask=lane0)
            plsc.store_scatter(ty1, [ridx], by1, mask=lane0)
            plsc.store_scatter(tx2, [ridx], bx2, mask=lane0)
            plsc.store_scatter(ty2, [ridx], by2, mask=lane0)
            plsc.store_scatter(
                tar, [ridx],
                (bx2 - bx1 + 1.0) * (by2 - by1 + 1.0), mask=lane0)
            return 0

        lax.fori_loop(0, PRE, gpick, 0)

        tkeep[pl.ds(0, 16)] = _splat_f(1.0)
        tkeep[pl.ds(16, 16)] = jnp.where(it + 16 < PRE, 1.0, 0.0)

        def nms(i, _):
            iidx = _splat_i(i)
            ki = plsc.load_gather(tkeep, [iidx])
            ai = plsc.load_gather(tar, [iidx])
            x1i = plsc.load_gather(tx1, [iidx])
            y1i = plsc.load_gather(ty1, [iidx])
            x2i = plsc.load_gather(tx2, [iidx])
            y2i = plsc.load_gather(ty2, [iidx])
            for c in range(2):
                sv = _splat_i(c * 16) + it
                a1 = tx1[pl.ds(c * 16, 16)]
                b1 = ty1[pl.ds(c * 16, 16)]
                a2 = tx2[pl.ds(c * 16, 16)]
                b2 = ty2[pl.ds(c * 16, 16)]
                ar = tar[pl.ds(c * 16, 16)]
                xx1 = jnp.maximum(x1i, a1)
                yy1 = jnp.maximum(y1i, b1)
                xx2 = jnp.minimum(x2i, a2)
                yy2 = jnp.minimum(y2i, b2)
                iw = jnp.maximum(xx2 - xx1 + 1.0, 0.0)
                ih = jnp.maximum(yy2 - yy1 + 1.0, 0.0)
                inter = iw * ih
                iou = inter / (ai + ar - inter)
                kv = tkeep[pl.ds(c * 16, 16)]
                sup = (iou > NMS_T) & (sv > iidx) & (ki > 0.0)
                tkeep[pl.ds(c * 16, 16)] = jnp.where(sup, 0.0, kv)
            return 0

        lax.fori_loop(0, PRE, nms, 0)

        for c in range(2):
            sv = _splat_i(c * 16) + it
            kv = tkeep[pl.ds(c * 16, 16)]
            fs = jnp.where(kv > 0.0, tsc2[pl.ds(c * 16, 16)], NEG)
            fsv[pl.ds(c * 16, 16)] = jnp.where(sv < PRE, fs, NEGINF)

        for c in range(4):
            stage[pl.ds(c * 16, 16)] = _splat_f(0.0)

        def outp(j, _):
            M = _splat_f(NEGINF)
            S = _splat_i(0)
            for k in range(2):
                v = fsv[pl.ds(k * 16, 16)]
                s = _splat_i(k * 16) + it
                better = (v > M) | ((v == M) & (s < S))
                M = jnp.where(better, v, M)
                S = jnp.where(better, s, S)
            m = _bcast_max(M)
            sidx = _bcast_min(jnp.where(M == m, S, 2147483647))
            plsc.store_scatter(fsv, [sidx], _splat_f(NEGINF), mask=lane0)
            bx1 = plsc.load_gather(tx1, [sidx])
            by1 = plsc.load_gather(ty1, [sidx])
            bx2 = plsc.load_gather(tx2, [sidx])
            by2 = plsc.load_gather(ty2, [sidx])
            val = jnp.where(it == 0, bx1,
                            jnp.where(it == 1, by1,
                                      jnp.where(it == 2, bx2, by2)))
            oidx = _splat_i(4 * j) + it
            plsc.store_scatter(stage, [oidx], val, mask=it < 4)
            return 0

        lax.fori_loop(0, POST, outp, 0)
        pltpu.sync_copy(stage, out_hbm)


def kernel(scores, bbox_deltas, im_info):
    fg = scores[0, A:].reshape(-1)
    bd = bbox_deltas[0].reshape(A, 4, HW)
    zpad = jnp.zeros((NPAD - N,), _f32)
    comps = [jnp.concatenate([fg, zpad])]
    for j in range(4):
        comps.append(jnp.concatenate([bd[:, j, :].reshape(-1), zpad]))
    xall = jnp.stack(comps).reshape(5, NWORK, SL).transpose(1, 0, 2).reshape(-1)
    imf = jnp.repeat(im_info.reshape(-1), 16)

    mesh = plsc.VectorSubcoreMesh(
        core_axis_name="c", subcore_axis_name="s", num_cores=1)
    vm = pltpu.VMEM
    out = pl.kernel(
        _sc_body,
        out_type=jax.ShapeDtypeStruct((64,), _f32),
        mesh=mesh,
        scratch_types=[
            vm((5 * SL,), _f32),
            vm((SL,), _f32), vm((SL,), _f32),
            vm((SL,), _f32), vm((SL,), _f32),
            vm((SL,), _f32), vm((SL,), _f32),
            vm((48,), _f32),
            vm((192,), _f32),
            pltpu.VMEM_SHARED((3072,), _f32),
            vm((3072,), _f32),
            vm((32,), _f32), vm((32,), _f32), vm((32,), _f32),
            vm((32,), _f32), vm((32,), _f32), vm((32,), _f32),
            vm((32,), _f32), vm((32,), _f32),
            vm((64,), _f32),
            pltpu.SemaphoreType.DMA,
        ],
        compiler_params=pltpu.CompilerParams(needs_layout_passes=False),
    )(xall, imf)
    return out[:40].reshape(POST, 4)
